# baseline (speedup 1.0000x reference)
import functools

import jax
import jax.numpy as jnp
from jax import lax
from jax.experimental import pallas as pl
from jax.experimental.pallas import tpu as pltpu
from jax.experimental.pallas import tpu_sc as plsc

_N, _T, _F = 1024, 512, 256
_B = 8
_NC, _NS = 2, 16
_NW = _NC * _NS
_NODES_PER_W = _N // _NW
_CH = 64
_NV = _F // 16


def _sc_body(x_hbm, meta_hbm, q_hbm, out_hbm, xb, qv, ov, mv):
    wid = lax.axis_index("s") * _NC + lax.axis_index("c")
    pltpu.sync_copy(meta_hbm, mv)
    pltpu.sync_copy(q_hbm, qv)
    lane = lax.iota(jnp.int32, 16)

    def per_node(i, carry):
        node = i * _NW + wid
        ptrv = mv[pl.ds(0, 16)]
        lenv = mv[pl.ds(16, 16)]
        gv = jnp.where((lane >= 1) & (lane <= _B - 1) & (ptrv <= node), 1, 0)
        g = jnp.sum(gv)
        node_len = jnp.sum(jnp.where(lane == g, lenv, 0))
        nch = (node_len + _CH - 1) // _CH

        def per_chunk(c, accs):
            pltpu.sync_copy(x_hbm.at[node, pl.ds(c * _CH, _CH), :], xb)
            valid = jnp.minimum(_CH, node_len - c * _CH)

            def per_row(t, accs):
                xs = [xb[t, pl.ds(16 * j, 16)] for j in range(_NV)]
                d = xs[0] * qv[pl.ds(0, 16)]
                for j in range(1, _NV):
                    d = d + xs[j] * qv[pl.ds(16 * j, 16)]
                dnums = lax.GatherDimensionNumbers(
                    offset_dims=(), collapsed_slice_dims=(0,),
                    start_index_map=(0,))
                for shift in (8, 4, 2, 1):
                    idx = jnp.bitwise_xor(lane, shift)
                    d = d + lax.gather(
                        d, idx[:, None], dnums, (1,),
                        mode=lax.GatherScatterMode.PROMISE_IN_BOUNDS)
                return tuple(accs[j] + xs[j] * d for j in range(_NV))

            return lax.fori_loop(0, valid, per_row, accs)

        accs = tuple(jnp.zeros((16,), jnp.float32) for _ in range(_NV))
        accs = lax.fori_loop(0, nch, per_chunk, accs)
        for j in range(_NV):
            ov[pl.ds(16 * j, 16)] = accs[j]
        pltpu.sync_copy(ov, out_hbm.at[node])
        return carry

    lax.fori_loop(0, _NODES_PER_W, per_node, jnp.int32(0))


def kernel(nodes_output, ptr, lengths, Wq_w):
    ptr_i = ptr.astype(jnp.int32)
    len_i = lengths.astype(jnp.int32)
    meta = jnp.full((32,), _N + _T, jnp.int32)
    meta = meta.at[0:_B + 1].set(ptr_i).at[16:16 + _B].set(len_i)

    mesh = plsc.VectorSubcoreMesh(core_axis_name="c", subcore_axis_name="s")
    run = functools.partial(
        pl.kernel,
        mesh=mesh,
        out_type=jax.ShapeDtypeStruct((_N, _F), jnp.float32),
        compiler_params=pltpu.CompilerParams(needs_layout_passes=False),
        scratch_types=[
            pltpu.VMEM((_CH, _F), jnp.float32),
            pltpu.VMEM((_F,), jnp.float32),
            pltpu.VMEM((_F,), jnp.float32),
            pltpu.VMEM((32,), jnp.int32),
        ],
    )(_sc_body)
    return run(nodes_output, meta, Wq_w)


# speedup vs baseline: 1.2547x; 1.2547x over previous
import functools

import jax
import jax.numpy as jnp
from jax import lax
from jax.experimental import pallas as pl
from jax.experimental.pallas import tpu as pltpu
from jax.experimental.pallas import tpu_sc as plsc

_N, _T, _F = 1024, 512, 256
_B = 8
_NC, _NS = 2, 16
_NW = _NC * _NS
_NODES_PER_W = _N // _NW
_CH = 64
_NV = _F // 16


def _sc_body(x_hbm, meta_hbm, q_hbm, out_hbm, xb0, xb1, qv, ov, mv,
             sem0, sem1):
    wid = lax.axis_index("s") * _NC + lax.axis_index("c")
    pltpu.sync_copy(meta_hbm, mv)
    pltpu.sync_copy(q_hbm, qv)
    lane = lax.iota(jnp.int32, 16)
    dnums = lax.GatherDimensionNumbers(
        offset_dims=(), collapsed_slice_dims=(0,), start_index_map=(0,))

    def per_node(i, carry):
        node = i * _NW + wid
        ptrv = mv[pl.ds(0, 16)]
        lenv = mv[pl.ds(16, 16)]
        gv = jnp.where((lane >= 1) & (lane <= _B - 1) & (ptrv <= node), 1, 0)
        g = jnp.sum(gv)
        node_len = jnp.sum(jnp.where(lane == g, lenv, 0))
        nch = (node_len + _CH - 1) // _CH

        pltpu.async_copy(x_hbm.at[node, pl.ds(0, _CH), :], xb0, sem0)

        @pl.when(nch > 1)
        def _prime1():
            pltpu.async_copy(x_hbm.at[node, pl.ds(_CH, _CH), :], xb1, sem1)

        def outer(k, accs):
            for b, (buf, sem) in enumerate(((xb0, sem0), (xb1, sem1))):
                c = 2 * k + b

                @pl.when(c < nch)
                def _wait():
                    pltpu.make_async_copy(
                        x_hbm.at[node, pl.ds(0, _CH), :], buf, sem).wait()

                rows = jnp.maximum(
                    0, jnp.minimum(_CH, node_len - c * _CH))

                def per_row(t, accs, buf=buf):
                    xs = [buf[t, pl.ds(16 * j, 16)] for j in range(_NV)]
                    d = xs[0] * qv[pl.ds(0, 16)]
                    for j in range(1, _NV):
                        d = d + xs[j] * qv[pl.ds(16 * j, 16)]
                    for shift in (8, 4, 2, 1):
                        idx = jnp.bitwise_xor(lane, shift)
                        d = d + lax.gather(
                            d, idx[:, None], dnums, (1,),
                            mode=lax.GatherScatterMode.PROMISE_IN_BOUNDS)
                    return tuple(accs[j] + xs[j] * d for j in range(_NV))

                accs = lax.fori_loop(0, rows, per_row, accs)

                @pl.when(c + 2 < nch)
                def _start_next():
                    pltpu.async_copy(
                        x_hbm.at[node, pl.ds((c + 2) * _CH, _CH), :], buf, sem)
            return accs

        accs = tuple(jnp.zeros((16,), jnp.float32) for _ in range(_NV))
        accs = lax.fori_loop(0, (nch + 1) // 2, outer, accs)
        for j in range(_NV):
            ov[pl.ds(16 * j, 16)] = accs[j]
        pltpu.sync_copy(ov, out_hbm.at[node])
        return carry

    lax.fori_loop(0, _NODES_PER_W, per_node, jnp.int32(0))


def kernel(nodes_output, ptr, lengths, Wq_w):
    ptr_i = ptr.astype(jnp.int32)
    len_i = lengths.astype(jnp.int32)
    meta = jnp.full((32,), _N + _T, jnp.int32)
    meta = meta.at[0:_B + 1].set(ptr_i).at[16:16 + _B].set(len_i)

    mesh = plsc.VectorSubcoreMesh(core_axis_name="c", subcore_axis_name="s")
    run = functools.partial(
        pl.kernel,
        mesh=mesh,
        out_type=jax.ShapeDtypeStruct((_N, _F), jnp.float32),
        compiler_params=pltpu.CompilerParams(needs_layout_passes=False),
        scratch_types=[
            pltpu.VMEM((_CH, _F), jnp.float32),
            pltpu.VMEM((_CH, _F), jnp.float32),
            pltpu.VMEM((_F,), jnp.float32),
            pltpu.VMEM((_F,), jnp.float32),
            pltpu.VMEM((32,), jnp.int32),
            pltpu.SemaphoreType.DMA,
            pltpu.SemaphoreType.DMA,
        ],
    )(_sc_body)
    return run(nodes_output, meta, Wq_w)


# speedup vs baseline: 1.3192x; 1.0514x over previous
import functools

import jax
import jax.numpy as jnp
from jax import lax
from jax.experimental import pallas as pl
from jax.experimental.pallas import tpu as pltpu
from jax.experimental.pallas import tpu_sc as plsc

_N, _T, _F = 1024, 512, 256
_B = 8
_NC, _NS = 2, 16
_NW = _NC * _NS
_NODES_PER_W = _N // _NW
_CH = 64
_NV = _F // 16


def _sc_body(x_hbm, meta_hbm, q_hbm, out_hbm, xb0, xb1, qv, ov, mv,
             sem0, sem1):
    wid = lax.axis_index("s") * _NC + lax.axis_index("c")
    pltpu.sync_copy(meta_hbm, mv)
    pltpu.sync_copy(q_hbm, qv)
    lane = lax.iota(jnp.int32, 16)
    dnums = lax.GatherDimensionNumbers(
        offset_dims=(), collapsed_slice_dims=(0,), start_index_map=(0,))

    def per_node(i, carry):
        node = i * _NW + wid
        ptrv = mv[pl.ds(0, 16)]
        lenv = mv[pl.ds(16, 16)]
        gv = jnp.where((lane >= 1) & (lane <= _B - 1) & (ptrv <= node), 1, 0)
        g = jnp.sum(gv)
        node_len = jnp.sum(jnp.where(lane == g, lenv, 0))
        nch = (node_len + _CH - 1) // _CH

        pltpu.async_copy(x_hbm.at[node, pl.ds(0, _CH), :], xb0, sem0)

        @pl.when(nch > 1)
        def _prime1():
            pltpu.async_copy(x_hbm.at[node, pl.ds(_CH, _CH), :], xb1, sem1)

        zero = jnp.zeros((16,), jnp.float32)
        for j in range(_NV):
            ov[pl.ds(16 * j, 16)] = zero

        def outer(k, qs):
            for b, (buf, sem) in enumerate(((xb0, sem0), (xb1, sem1))):
                c = 2 * k + b

                @pl.when(c < nch)
                def _wait():
                    pltpu.make_async_copy(
                        x_hbm.at[node, pl.ds(0, _CH), :], buf, sem).wait()

                rows = jnp.maximum(
                    0, jnp.minimum(_CH, node_len - c * _CH))

                def per_row(t, qs, buf=buf):
                    xs = [buf[t, pl.ds(16 * j, 16)] for j in range(_NV)]
                    d = xs[0] * qs[0]
                    for j in range(1, _NV):
                        d = d + xs[j] * qs[j]
                    for shift in (8, 4, 2, 1):
                        idx = jnp.bitwise_xor(lane, shift)
                        d = d + lax.gather(
                            d, idx[:, None], dnums, (1,),
                            mode=lax.GatherScatterMode.PROMISE_IN_BOUNDS)
                    for j in range(_NV):
                        plsc.addupdate(ov.at[pl.ds(16 * j, 16)], xs[j] * d)
                    return qs

                qs = lax.fori_loop(0, rows, per_row, qs)

                @pl.when(c + 2 < nch)
                def _start_next():
                    pltpu.async_copy(
                        x_hbm.at[node, pl.ds((c + 2) * _CH, _CH), :], buf, sem)
            return qs

        qs = tuple(qv[pl.ds(16 * j, 16)] for j in range(_NV))
        lax.fori_loop(0, (nch + 1) // 2, outer, qs)
        pltpu.sync_copy(ov, out_hbm.at[node])
        return carry

    lax.fori_loop(0, _NODES_PER_W, per_node, jnp.int32(0))


def kernel(nodes_output, ptr, lengths, Wq_w):
    ptr_i = ptr.astype(jnp.int32)
    len_i = lengths.astype(jnp.int32)
    meta = jnp.full((32,), _N + _T, jnp.int32)
    meta = meta.at[0:_B + 1].set(ptr_i).at[16:16 + _B].set(len_i)

    mesh = plsc.VectorSubcoreMesh(core_axis_name="c", subcore_axis_name="s")
    run = functools.partial(
        pl.kernel,
        mesh=mesh,
        out_type=jax.ShapeDtypeStruct((_N, _F), jnp.float32),
        compiler_params=pltpu.CompilerParams(needs_layout_passes=False),
        scratch_types=[
            pltpu.VMEM((_CH, _F), jnp.float32),
            pltpu.VMEM((_CH, _F), jnp.float32),
            pltpu.VMEM((_F,), jnp.float32),
            pltpu.VMEM((_F,), jnp.float32),
            pltpu.VMEM((32,), jnp.int32),
            pltpu.SemaphoreType.DMA,
            pltpu.SemaphoreType.DMA,
        ],
    )(_sc_body)
    return run(nodes_output, meta, Wq_w)


# speedup vs baseline: 1.9150x; 1.4516x over previous
import functools

import jax
import jax.numpy as jnp
from jax import lax
from jax.experimental import pallas as pl
from jax.experimental.pallas import tpu as pltpu
from jax.experimental.pallas import tpu_sc as plsc

_N, _T, _F = 1024, 512, 256
_B = 8
_NC, _NS = 2, 16
_NW = _NC * _NS
_NODES_PER_W = _N // _NW
_CH = 64
_NV = _F // 16


def _sc_body(x_hbm, meta_hbm, q_hbm, out_hbm, xb0, xb1, qv, ov, mv,
             sem0, sem1):
    wid = lax.axis_index("s") * _NC + lax.axis_index("c")
    pltpu.sync_copy(meta_hbm, mv)
    pltpu.sync_copy(q_hbm, qv)
    lane = lax.iota(jnp.int32, 16)
    dnums = lax.GatherDimensionNumbers(
        offset_dims=(), collapsed_slice_dims=(0,), start_index_map=(0,))

    def per_node(i, carry):
        node = i * _NW + wid
        ptrv = mv[pl.ds(0, 16)]
        lenv = mv[pl.ds(16, 16)]
        gv = jnp.where((lane >= 1) & (lane <= _B - 1) & (ptrv <= node), 1, 0)
        g = jnp.sum(gv)
        node_len = jnp.sum(jnp.where(lane == g, lenv, 0))
        nch = (node_len + _CH - 1) // _CH

        pltpu.async_copy(x_hbm.at[node, pl.ds(0, _CH), :], xb0, sem0)

        @pl.when(nch > 1)
        def _prime1():
            pltpu.async_copy(x_hbm.at[node, pl.ds(_CH, _CH), :], xb1, sem1)

        zero = jnp.zeros((16,), jnp.float32)
        for j in range(_NV):
            ov[pl.ds(16 * j, 16)] = zero

        def outer(k, qs):
            for b, (buf, sem) in enumerate(((xb0, sem0), (xb1, sem1))):
                c = 2 * k + b

                @pl.when(c < nch)
                def _wait():
                    pltpu.make_async_copy(
                        x_hbm.at[node, pl.ds(0, _CH), :], buf, sem).wait()

                rows = jnp.maximum(
                    0, jnp.minimum(_CH, node_len - c * _CH))

                def per_row(t, qs, buf=buf):
                    xs = [buf[t, pl.ds(16 * j, 16)] for j in range(_NV)]
                    ps = [xs[j] * qs[j] for j in range(_NV)]
                    while len(ps) > 1:
                        ps = [ps[m] + ps[m + 1] for m in range(0, len(ps), 2)]
                    d = ps[0]
                    for shift in (8, 4, 2, 1):
                        idx = jnp.bitwise_xor(lane, shift)
                        d = d + lax.gather(
                            d, idx[:, None], dnums, (1,),
                            mode=lax.GatherScatterMode.PROMISE_IN_BOUNDS)
                    for j in range(_NV):
                        plsc.addupdate(ov.at[pl.ds(16 * j, 16)], xs[j] * d)
                    return qs

                qs = plsc.parallel_loop(0, rows, 1, unroll=4, carry=qs)(per_row)

                @pl.when(c + 2 < nch)
                def _start_next():
                    pltpu.async_copy(
                        x_hbm.at[node, pl.ds((c + 2) * _CH, _CH), :], buf, sem)
            return qs

        qs = tuple(qv[pl.ds(16 * j, 16)] for j in range(_NV))
        lax.fori_loop(0, (nch + 1) // 2, outer, qs)
        pltpu.sync_copy(ov, out_hbm.at[node])
        return carry

    lax.fori_loop(0, _NODES_PER_W, per_node, jnp.int32(0))


def kernel(nodes_output, ptr, lengths, Wq_w):
    ptr_i = ptr.astype(jnp.int32)
    len_i = lengths.astype(jnp.int32)
    meta = jnp.full((32,), _N + _T, jnp.int32)
    meta = meta.at[0:_B + 1].set(ptr_i).at[16:16 + _B].set(len_i)

    mesh = plsc.VectorSubcoreMesh(core_axis_name="c", subcore_axis_name="s")
    run = functools.partial(
        pl.kernel,
        mesh=mesh,
        out_type=jax.ShapeDtypeStruct((_N, _F), jnp.float32),
        compiler_params=pltpu.CompilerParams(needs_layout_passes=False),
        scratch_types=[
            pltpu.VMEM((_CH, _F), jnp.float32),
            pltpu.VMEM((_CH, _F), jnp.float32),
            pltpu.VMEM((_F,), jnp.float32),
            pltpu.VMEM((_F,), jnp.float32),
            pltpu.VMEM((32,), jnp.int32),
            pltpu.SemaphoreType.DMA,
            pltpu.SemaphoreType.DMA,
        ],
    )(_sc_body)
    return run(nodes_output, meta, Wq_w)
